# Initial kernel scaffold; baseline (speedup 1.0000x reference)
import jax
import jax.numpy as jnp
from jax.experimental import pallas as pl


def kernel(x, gate_W, expert_W, out_W):
    raise NotImplementedError("write your pallas kernel here")


# speedup vs baseline: 1.5234x; 1.5234x over previous
import jax
import jax.numpy as jnp
from jax.experimental import pallas as pl
from jax.experimental.pallas import tpu as pltpu

_INPUT_DIM = 1024
_INTER_DIM = 2048
_GATE_NUM = 8
_TOP_P = 0.3

_TM = 256


def _fused_moe_kernel(x_ref, gate_w_ref, w01_ref, out_wt_ref, o_ref):
    x = x_ref[...]

    g = jax.lax.dot_general(
        x, gate_w_ref[...],
        dimension_numbers=(((1,), (1,)), ((), ())),
        preferred_element_type=jnp.float32,
        precision=jax.lax.Precision.HIGHEST,
    )
    m = jnp.max(g, axis=1, keepdims=True)
    e = jnp.exp(g - m)
    probs = e / jnp.sum(e, axis=1, keepdims=True)
    c1 = jnp.sum((probs > _TOP_P).astype(jnp.float32), axis=1)
    s0 = probs[:, 0] * (_GATE_NUM - c1)
    s1 = probs[:, 1] * c1

    xb = x.astype(jnp.bfloat16)
    h = jax.lax.dot_general(
        xb, w01_ref[...],
        dimension_numbers=(((1,), (0,)), ((), ())),
        preferred_element_type=jnp.float32,
    )
    fh = h[:, :_INTER_DIM] * s0[:, None] + h[:, _INTER_DIM:] * s1[:, None]

    o_ref[...] = jax.lax.dot_general(
        fh.astype(jnp.bfloat16), out_wt_ref[...],
        dimension_numbers=(((1,), (0,)), ((), ())),
        preferred_element_type=jnp.float32,
    )


def kernel(x, gate_W, expert_W, out_W):
    bsz, seql, embs = x.shape
    n = bsz * seql
    x_flat = x.reshape(n, embs)
    w01_t = (
        jnp.concatenate([expert_W[0], expert_W[1]], axis=0)
        .T.astype(jnp.bfloat16)
    )
    out_wt = out_W.T.astype(jnp.bfloat16)

    grid = (n // _TM,)
    out = pl.pallas_call(
        _fused_moe_kernel,
        grid=grid,
        in_specs=[
            pl.BlockSpec((_TM, embs), lambda i: (i, 0)),
            pl.BlockSpec((_GATE_NUM, embs), lambda i: (0, 0)),
            pl.BlockSpec((embs, 2 * _INTER_DIM), lambda i: (0, 0)),
            pl.BlockSpec((_INTER_DIM, embs), lambda i: (0, 0)),
        ],
        out_specs=pl.BlockSpec((_TM, embs), lambda i: (i, 0)),
        out_shape=jax.ShapeDtypeStruct((n, embs), jnp.float32),
        compiler_params=pltpu.CompilerParams(
            dimension_semantics=("arbitrary",),
        ),
    )(x_flat, gate_W, w01_t, out_wt)
    return out.reshape(bsz, seql, embs)


# speedup vs baseline: 1.5244x; 1.0006x over previous
import jax
import jax.numpy as jnp
from jax.experimental import pallas as pl
from jax.experimental.pallas import tpu as pltpu

_INPUT_DIM = 1024
_INTER_DIM = 2048
_GATE_NUM = 8
_TOP_P = 0.3

_TM = 256


def _fused_moe_kernel(x_ref, gate_w_ref, w01_ref, out_wt_ref, o_ref):
    x = x_ref[...]

    g = jax.lax.dot_general(
        x, gate_w_ref[...],
        dimension_numbers=(((1,), (1,)), ((), ())),
        preferred_element_type=jnp.float32,
        precision=jax.lax.Precision.HIGHEST,
    )
    m = jnp.max(g, axis=1, keepdims=True)
    e = jnp.exp(g - m)
    probs = e / jnp.sum(e, axis=1, keepdims=True)
    c1 = jnp.sum((probs > _TOP_P).astype(jnp.float32), axis=1)
    s0 = probs[:, 0] * (_GATE_NUM - c1)
    s1 = probs[:, 1] * c1

    xb = x.astype(jnp.bfloat16)
    h = jax.lax.dot_general(
        xb, w01_ref[...],
        dimension_numbers=(((1,), (0,)), ((), ())),
        preferred_element_type=jnp.float32,
    )
    fh = h[:, :_INTER_DIM] * s0[:, None] + h[:, _INTER_DIM:] * s1[:, None]

    o_ref[...] = jax.lax.dot_general(
        fh.astype(jnp.bfloat16), out_wt_ref[...],
        dimension_numbers=(((1,), (0,)), ((), ())),
        preferred_element_type=jnp.float32,
    )


def kernel(x, gate_W, expert_W, out_W):
    bsz, seql, embs = x.shape
    n = bsz * seql
    x_flat = x.reshape(n, embs)
    w01_t = (
        jnp.concatenate([expert_W[0], expert_W[1]], axis=0)
        .T.astype(jnp.bfloat16)
    )
    out_wt = out_W.T.astype(jnp.bfloat16)

    grid = (n // _TM,)
    out = pl.pallas_call(
        _fused_moe_kernel,
        grid=grid,
        in_specs=[
            pl.BlockSpec((_TM, embs), lambda i: (i, 0)),
            pl.BlockSpec((_GATE_NUM, embs), lambda i: (0, 0)),
            pl.BlockSpec((embs, 2 * _INTER_DIM), lambda i: (0, 0)),
            pl.BlockSpec((_INTER_DIM, embs), lambda i: (0, 0)),
        ],
        out_specs=pl.BlockSpec((_TM, embs), lambda i: (i, 0)),
        out_shape=jax.ShapeDtypeStruct((n, embs), jnp.float32),
        compiler_params=pltpu.CompilerParams(
            dimension_semantics=("parallel",),
        ),
    )(x_flat, gate_W, w01_t, out_wt)
    return out.reshape(bsz, seql, embs)


# speedup vs baseline: 3.7385x; 2.4524x over previous
import jax
import jax.numpy as jnp
from jax.experimental import pallas as pl
from jax.experimental.pallas import tpu as pltpu

_INPUT_DIM = 1024
_INTER_DIM = 2048
_GATE_NUM = 8
_TOP_P = 0.3

_TM = 256


def _fused_moe_kernel(x_ref, gate_w_ref, ew_ref, out_w_ref, o_ref):
    x = x_ref[...]

    g = jax.lax.dot_general(
        x, gate_w_ref[...],
        dimension_numbers=(((1,), (1,)), ((), ())),
        preferred_element_type=jnp.float32,
        precision=jax.lax.Precision.HIGHEST,
    )
    m = jnp.max(g, axis=1, keepdims=True)
    e = jnp.exp(g - m)
    probs = e / jnp.sum(e, axis=1, keepdims=True)
    c1 = jnp.sum((probs > _TOP_P).astype(jnp.float32), axis=1)
    s0 = probs[:, 0] * (_GATE_NUM - c1)
    s1 = probs[:, 1] * c1

    xb = x.astype(jnp.bfloat16)
    w01 = ew_ref[...].reshape(2 * _INTER_DIM, _INPUT_DIM).astype(jnp.bfloat16)
    h = jax.lax.dot_general(
        xb, w01,
        dimension_numbers=(((1,), (1,)), ((), ())),
        preferred_element_type=jnp.float32,
    )
    fh = h[:, :_INTER_DIM] * s0[:, None] + h[:, _INTER_DIM:] * s1[:, None]

    o_ref[...] = jax.lax.dot_general(
        fh.astype(jnp.bfloat16), out_w_ref[...].astype(jnp.bfloat16),
        dimension_numbers=(((1,), (1,)), ((), ())),
        preferred_element_type=jnp.float32,
    )


def kernel(x, gate_W, expert_W, out_W):
    bsz, seql, embs = x.shape
    n = bsz * seql
    x_flat = x.reshape(n, embs)

    grid = (n // _TM,)
    out = pl.pallas_call(
        _fused_moe_kernel,
        grid=grid,
        in_specs=[
            pl.BlockSpec((_TM, embs), lambda i: (i, 0)),
            pl.BlockSpec((_GATE_NUM, embs), lambda i: (0, 0)),
            pl.BlockSpec((2, _INTER_DIM, embs), lambda i: (0, 0, 0)),
            pl.BlockSpec((embs, _INTER_DIM), lambda i: (0, 0)),
        ],
        out_specs=pl.BlockSpec((_TM, embs), lambda i: (i, 0)),
        out_shape=jax.ShapeDtypeStruct((n, embs), jnp.float32),
        compiler_params=pltpu.CompilerParams(
            dimension_semantics=("parallel",),
        ),
    )(x_flat, gate_W, expert_W, out_W)
    return out.reshape(bsz, seql, embs)


# speedup vs baseline: 4.3365x; 1.1600x over previous
import jax
import jax.numpy as jnp
from jax.experimental import pallas as pl
from jax.experimental.pallas import tpu as pltpu

_INPUT_DIM = 1024
_INTER_DIM = 2048
_GATE_NUM = 8
_TOP_P = 0.3

_TM = 256


def _fused_moe_kernel(x_ref, gate_w_ref, ew_ref, out_w_ref, o_ref):
    x = x_ref[...]

    g = jax.lax.dot_general(
        x, gate_w_ref[...],
        dimension_numbers=(((1,), (1,)), ((), ())),
        preferred_element_type=jnp.float32,
        precision=jax.lax.Precision.DEFAULT,
    )
    m = jnp.max(g, axis=1, keepdims=True)
    e = jnp.exp(g - m)
    probs = e / jnp.sum(e, axis=1, keepdims=True)
    c1 = jnp.sum((probs > _TOP_P).astype(jnp.float32), axis=1)
    s0 = probs[:, 0] * (_GATE_NUM - c1)
    s1 = probs[:, 1] * c1

    xb = x.astype(jnp.bfloat16)
    w01 = ew_ref[...].reshape(2 * _INTER_DIM, _INPUT_DIM).astype(jnp.bfloat16)
    h = jax.lax.dot_general(
        xb, w01,
        dimension_numbers=(((1,), (1,)), ((), ())),
        preferred_element_type=jnp.float32,
    )
    fh = h[:, :_INTER_DIM] * s0[:, None] + h[:, _INTER_DIM:] * s1[:, None]

    o_ref[...] = jax.lax.dot_general(
        fh.astype(jnp.bfloat16), out_w_ref[...].astype(jnp.bfloat16),
        dimension_numbers=(((1,), (1,)), ((), ())),
        preferred_element_type=jnp.float32,
    )


def kernel(x, gate_W, expert_W, out_W):
    bsz, seql, embs = x.shape
    n = bsz * seql
    x_flat = x.reshape(n, embs)

    grid = (n // _TM,)
    out = pl.pallas_call(
        _fused_moe_kernel,
        grid=grid,
        in_specs=[
            pl.BlockSpec((_TM, embs), lambda i: (i, 0)),
            pl.BlockSpec((_GATE_NUM, embs), lambda i: (0, 0)),
            pl.BlockSpec((2, _INTER_DIM, embs), lambda i: (0, 0, 0)),
            pl.BlockSpec((embs, _INTER_DIM), lambda i: (0, 0)),
        ],
        out_specs=pl.BlockSpec((_TM, embs), lambda i: (i, 0)),
        out_shape=jax.ShapeDtypeStruct((n, embs), jnp.float32),
        compiler_params=pltpu.CompilerParams(
            dimension_semantics=("parallel",),
        ),
    )(x_flat, gate_W, expert_W, out_W)
    return out.reshape(bsz, seql, embs)
